# baseline (speedup 1.0000x reference)
import functools

import jax
import jax.numpy as jnp
from jax import lax
from jax.experimental import pallas as pl
from jax.experimental.pallas import tpu as pltpu
from jax.experimental.pallas import tpu_sc as plsc

_NC = 2
_NS = 16
_NW = _NC * _NS
_CH = 32


def _sc_body(x_hbm, emb_hbm, out_hbm, ebuf, xbuf):
    B, S, D = x_hbm.shape
    wid = lax.axis_index("s") * _NC + lax.axis_index("c")
    s_per_w = S // _NW
    s_base = wid * s_per_w
    n_chunks = s_per_w // _CH

    def chunk_body(c, carry):
        s0 = s_base + c * _CH
        pltpu.sync_copy(emb_hbm.at[pl.ds(s0, _CH), :], ebuf)
        for b in range(B):
            pltpu.sync_copy(x_hbm.at[b, pl.ds(s0, _CH), :], xbuf)

            def row_body(i, rcarry):
                for j in range(D // 16):
                    sl = pl.ds(j * 16, 16)
                    plsc.addupdate(xbuf.at[i, sl], ebuf[i, sl])
                return rcarry

            lax.fori_loop(0, _CH, row_body, 0)
            pltpu.sync_copy(xbuf, out_hbm.at[b, pl.ds(s0, _CH), :])
        return carry

    lax.fori_loop(0, n_chunks, chunk_body, 0)


def kernel(x, emb):
    B, S, D = x.shape
    mesh = plsc.VectorSubcoreMesh(core_axis_name="c", subcore_axis_name="s")
    run = pl.kernel(
        _sc_body,
        mesh=mesh,
        out_type=jax.ShapeDtypeStruct((B, S, D), x.dtype),
        scratch_types=[
            pltpu.VMEM((_CH, D), jnp.float32),
            pltpu.VMEM((_CH, D), jnp.float32),
        ],
    )
    return run(x, emb)
